# baseline (speedup 1.0000x reference)
import functools

import jax
import jax.numpy as jnp
from jax.experimental import pallas as pl

N_FEATS = 9
VOCAB = 64
DIM_PE = 32
MAX_FREQS = 16


def _body(x_ref, pe_ref, thi_ref, tlo_ref, wbig_ref, bbig_ref, w1bd_ref,
          b1t_ref, wpp_ref, bpp_ref, out_ref):
    B = x_ref.shape[0]
    xb = x_ref[...]
    oh = jnp.concatenate(
        [(xb[:, i:i + 1] == jax.lax.broadcasted_iota(jnp.int32, (B, VOCAB), 1)
          ).astype(jnp.bfloat16) for i in range(N_FEATS)], axis=1)
    h_pad = (jnp.dot(oh, thi_ref[...], preferred_element_type=jnp.float32)
             + jnp.dot(oh, tlo_ref[...], preferred_element_type=jnp.float32))

    xpe = pe_ref[...]
    pos = jax.nn.relu(jnp.dot(xpe, wbig_ref[...],
                              preferred_element_type=jnp.float32) + bbig_ref[...])
    s = jax.nn.relu(jnp.dot(pos.astype(jnp.bfloat16), w1bd_ref[...],
                            preferred_element_type=jnp.float32) + b1t_ref[...])
    pe_pad = jax.nn.relu(jnp.dot(s.astype(jnp.bfloat16), wpp_ref[...],
                                 preferred_element_type=jnp.float32) + bpp_ref[...])
    out_ref[...] = h_pad + pe_pad


def kernel(x, pestat, emb_tables, W_A, b_A, W1, b1, W_post, b_post):
    N = x.shape[0]
    D1 = emb_tables.shape[-1]
    EMB = D1 + DIM_PE
    F = MAX_FREQS

    tables_flat = emb_tables.reshape(N_FEATS * VOCAB, D1)
    tpad = jnp.pad(tables_flat, ((0, 0), (0, EMB - D1)))
    thi = tpad.astype(jnp.bfloat16)
    tlo = (tpad - thi.astype(jnp.float32)).astype(jnp.bfloat16)

    eye_f = jnp.eye(F, dtype=jnp.float32)
    wbig = jnp.zeros((2 * F, F * DIM_PE), jnp.float32)
    wbig = wbig.at[0::2, :].set(jnp.kron(eye_f, W_A[0:1, :]))
    wbig = wbig.at[1::2, :].set(jnp.kron(eye_f, W_A[1:2, :]))
    bbig = jnp.tile(b_A, (F,))[None, :]

    w1bd = jnp.kron(eye_f, W1).astype(jnp.bfloat16)
    b1t = jnp.tile(b1, (F,))[None, :]

    wpp = jnp.zeros((F * DIM_PE, EMB), jnp.float32)
    wpp = wpp.at[:, D1:].set(jnp.tile(W_post, (F, 1))).astype(jnp.bfloat16)
    bpp = jnp.zeros((1, EMB), jnp.float32).at[0, D1:].set(b_post)

    xpe = pestat.reshape(N, 2 * F)

    B = 512
    grid = (pl.cdiv(N, B),)
    full = lambda shape: pl.BlockSpec(shape, lambda i: (0,) * len(shape))
    out = pl.pallas_call(
        _body,
        grid=grid,
        in_specs=[
            pl.BlockSpec((B, N_FEATS), lambda i: (i, 0)),
            pl.BlockSpec((B, 2 * F), lambda i: (i, 0)),
            full(thi.shape), full(tlo.shape), full(wbig.shape), full(bbig.shape),
            full(w1bd.shape), full(b1t.shape), full(wpp.shape), full(bpp.shape),
        ],
        out_specs=pl.BlockSpec((B, EMB), lambda i: (i, 0)),
        out_shape=jax.ShapeDtypeStruct((N, EMB), jnp.float32),
    )(x.astype(jnp.int32), xpe, thi, tlo, wbig, bbig, w1bd, b1t, wpp, bpp)
    return out


# speedup vs baseline: 1.8686x; 1.8686x over previous
import functools

import jax
import jax.numpy as jnp
from jax.experimental import pallas as pl

N_FEATS = 9
VOCAB = 64
DIM_PE = 32
MAX_FREQS = 16


def _body(x_ref, pe_ref, thi_ref, tlo_ref, wbig_ref, bbig_ref, w1bd_ref,
          b1t_ref, wpp_ref, bpp_ref, out_ref):
    B = x_ref.shape[0]
    xb = x_ref[...]
    oh = jnp.concatenate(
        [(xb[:, i:i + 1] == jax.lax.broadcasted_iota(jnp.int32, (B, VOCAB), 1)
          ).astype(jnp.float32) for i in range(N_FEATS)], axis=1)
    h_pad = jnp.dot(oh, thi_ref[...], preferred_element_type=jnp.float32)

    xpe = pe_ref[...]
    pos = jax.nn.relu(jnp.dot(xpe, wbig_ref[...],
                              preferred_element_type=jnp.float32) + bbig_ref[...])
    s = jax.nn.relu(jnp.dot(pos.astype(jnp.bfloat16), w1bd_ref[...],
                            preferred_element_type=jnp.float32) + b1t_ref[...])
    pe_pad = jax.nn.relu(jnp.dot(s.astype(jnp.bfloat16), wpp_ref[...],
                                 preferred_element_type=jnp.float32) + bpp_ref[...])
    out_ref[...] = h_pad + pe_pad


def kernel(x, pestat, emb_tables, W_A, b_A, W1, b1, W_post, b_post):
    N = x.shape[0]
    D1 = emb_tables.shape[-1]
    EMB = D1 + DIM_PE
    F = MAX_FREQS

    tables_flat = emb_tables.reshape(N_FEATS * VOCAB, D1)
    tpad = jnp.pad(tables_flat, ((0, 0), (0, EMB - D1)))
    thi = tpad
    tlo = tpad[:1]

    eye_f = jnp.eye(F, dtype=jnp.float32)
    wbig = jnp.zeros((2 * F, F * DIM_PE), jnp.float32)
    wbig = wbig.at[0::2, :].set(jnp.kron(eye_f, W_A[0:1, :]))
    wbig = wbig.at[1::2, :].set(jnp.kron(eye_f, W_A[1:2, :]))
    bbig = jnp.tile(b_A, (F,))[None, :]

    w1bd = jnp.kron(eye_f, W1).astype(jnp.bfloat16)
    b1t = jnp.tile(b1, (F,))[None, :]

    wpp = jnp.zeros((F * DIM_PE, EMB), jnp.float32)
    wpp = wpp.at[:, D1:].set(jnp.tile(W_post, (F, 1))).astype(jnp.bfloat16)
    bpp = jnp.zeros((1, EMB), jnp.float32).at[0, D1:].set(b_post)

    xpe = pestat.reshape(N, 2 * F)

    B = 512
    grid = (pl.cdiv(N, B),)
    full = lambda shape: pl.BlockSpec(shape, lambda i: (0,) * len(shape))
    out = pl.pallas_call(
        _body,
        grid=grid,
        in_specs=[
            pl.BlockSpec((B, N_FEATS), lambda i: (i, 0)),
            pl.BlockSpec((B, 2 * F), lambda i: (i, 0)),
            full(thi.shape), full(tlo.shape), full(wbig.shape), full(bbig.shape),
            full(w1bd.shape), full(b1t.shape), full(wpp.shape), full(bpp.shape),
        ],
        out_specs=pl.BlockSpec((B, EMB), lambda i: (i, 0)),
        out_shape=jax.ShapeDtypeStruct((N, EMB), jnp.float32),
    )(x.astype(jnp.int32), xpe, thi, tlo, wbig, bbig, w1bd, b1t, wpp, bpp)
    return out


# speedup vs baseline: 2.0406x; 1.0921x over previous
import functools

import jax
import jax.numpy as jnp
from jax.experimental import pallas as pl

N_FEATS = 9
VOCAB = 64
DIM_PE = 32
MAX_FREQS = 16


def _body(x_ref, pe_ref, thi_ref, tlo_ref, wbig_ref, bbig_ref, w1bd_ref,
          b1t_ref, wpp_ref, bpp_ref, out_ref):
    B = x_ref.shape[0]
    xb = x_ref[...]
    oh = jnp.concatenate(
        [(xb[:, i:i + 1] == jax.lax.broadcasted_iota(jnp.int32, (B, VOCAB), 1)
          ).astype(jnp.float32) for i in range(N_FEATS)], axis=1)
    h_pad = jnp.dot(oh, thi_ref[...], preferred_element_type=jnp.float32)

    xpe = pe_ref[...].astype(jnp.bfloat16)
    pos = jax.nn.relu(jnp.dot(xpe, wbig_ref[...],
                              preferred_element_type=jnp.float32) + bbig_ref[...])
    s = jax.nn.relu(jnp.dot(pos.astype(jnp.bfloat16), w1bd_ref[...],
                            preferred_element_type=jnp.float32) + b1t_ref[...])
    pe_pad = jax.nn.relu(jnp.dot(s.astype(jnp.bfloat16), wpp_ref[...],
                                 preferred_element_type=jnp.float32) + bpp_ref[...])
    out_ref[...] = h_pad + pe_pad


def kernel(x, pestat, emb_tables, W_A, b_A, W1, b1, W_post, b_post):
    N = x.shape[0]
    D1 = emb_tables.shape[-1]
    EMB = D1 + DIM_PE
    F = MAX_FREQS

    tables_flat = emb_tables.reshape(N_FEATS * VOCAB, D1)
    tpad = jnp.pad(tables_flat, ((0, 0), (0, EMB - D1)))
    thi = tpad
    tlo = tpad[:1]

    eye_f = jnp.eye(F, dtype=jnp.float32)
    wbig = jnp.zeros((2 * F, F * DIM_PE), jnp.float32)
    wbig = wbig.at[0::2, :].set(jnp.kron(eye_f, W_A[0:1, :]))
    wbig = wbig.at[1::2, :].set(jnp.kron(eye_f, W_A[1:2, :])).astype(jnp.bfloat16)
    bbig = jnp.tile(b_A, (F,))[None, :]

    w1bd = jnp.kron(eye_f, W1).astype(jnp.bfloat16)
    b1t = jnp.tile(b1, (F,))[None, :]

    wpp = jnp.zeros((F * DIM_PE, EMB), jnp.float32)
    wpp = wpp.at[:, D1:].set(jnp.tile(W_post, (F, 1))).astype(jnp.bfloat16)
    bpp = jnp.zeros((1, EMB), jnp.float32).at[0, D1:].set(b_post)

    xpe = pestat.reshape(N, 2 * F)

    B = 1024
    grid = (pl.cdiv(N, B),)
    full = lambda shape: pl.BlockSpec(shape, lambda i: (0,) * len(shape))
    out = pl.pallas_call(
        _body,
        grid=grid,
        in_specs=[
            pl.BlockSpec((B, N_FEATS), lambda i: (i, 0)),
            pl.BlockSpec((B, 2 * F), lambda i: (i, 0)),
            full(thi.shape), full(tlo.shape), full(wbig.shape), full(bbig.shape),
            full(w1bd.shape), full(b1t.shape), full(wpp.shape), full(bpp.shape),
        ],
        out_specs=pl.BlockSpec((B, EMB), lambda i: (i, 0)),
        out_shape=jax.ShapeDtypeStruct((N, EMB), jnp.float32),
    )(x.astype(jnp.int32), xpe, thi, tlo, wbig, bbig, w1bd, b1t, wpp, bpp)
    return out
